# baseline (speedup 1.0000x reference)
import functools

import jax
import jax.numpy as jnp
from jax import lax
from jax.experimental import pallas as pl
from jax.experimental.pallas import tpu as pltpu
from jax.experimental.pallas import tpu_sc as plsc

N_NODES = 10000
N_EDGES = 320000
D = 128
NUM_RELS = 16

NUM_CORES = 2
NUM_SUBCORES = 16
NW = NUM_CORES * NUM_SUBCORES
B_CH = 128
CH_PER_W = 80
CH_PH = 40
NUM_CHUNKS = NW * CH_PER_W
E_PAD = NUM_CHUNKS * B_CH
N_PAD = 10240
ROWS_PER_SUB = N_PAD // NUM_SUBCORES


def _xw_body(feat_ref, w_ref, xw_ref):
    xw_ref[0] = jnp.dot(feat_ref[...], w_ref[0],
                        preferred_element_type=jnp.float32)


def _make_xw(feat, weight):
    bn = 2000
    nb = N_NODES // bn
    return pl.pallas_call(
        _xw_body,
        grid=(nb, NUM_RELS),
        in_specs=[
            pl.BlockSpec((bn, D), lambda i, r: (i, 0)),
            pl.BlockSpec((1, D, D), lambda i, r: (r, 0, 0)),
        ],
        out_specs=pl.BlockSpec((1, bn, D), lambda i, r: (r, i, 0)),
        out_shape=jax.ShapeDtypeStruct((NUM_RELS, N_NODES, D), jnp.float32),
    )(feat, weight)


def _sc_body(xw_hbm, src_hbm, et_hbm, dst_hbm, zeros_hbm, part_hbm,
             et_a, et_b, dst_a, dst_b, idx_a, idx_b, rows_a,
             acc_shared, sem_a, sem_ia, sem_ib):
    core = lax.axis_index("c")
    sub = lax.axis_index("s")
    wid = core * NUM_SUBCORES + sub

    r0 = sub * ROWS_PER_SUB
    pltpu.sync_copy(zeros_hbm.at[pl.ds(r0, ROWS_PER_SUB)],
                    acc_shared.at[pl.ds(r0, ROWS_PER_SUB)])
    plsc.subcore_barrier()

    @pl.loop(0, CH_PER_W)
    def _(j):
        c = j * NW + wid
        base = pl.multiple_of(c * B_CH, B_CH)
        pltpu.sync_copy(src_hbm.at[pl.ds(base, B_CH)], et_b)
        pltpu.sync_copy(et_hbm.at[pl.ds(base, B_CH)], et_a)
        pltpu.sync_copy(dst_hbm.at[pl.ds(base, B_CH)], dst_a)
        for k in range(B_CH // 16):
            sl = pl.ds(k * 16, 16)
            idx_a[sl] = et_a[sl] * N_NODES + et_b[sl]
        pltpu.async_copy(xw_hbm.at[idx_a], rows_a, sem_a).wait()
        pltpu.sync_copy(rows_a, acc_shared.at[dst_a], add=True)

    plsc.subcore_barrier()
    pltpu.sync_copy(acc_shared.at[pl.ds(r0, ROWS_PER_SUB)],
                    part_hbm.at[core, pl.ds(r0, ROWS_PER_SUB)])


def _run_sc(xw_flat, src, et, dst, zeros):
    mesh = plsc.VectorSubcoreMesh(core_axis_name="c", subcore_axis_name="s")
    k = pl.kernel(
        _sc_body,
        out_type=jax.ShapeDtypeStruct((NUM_CORES, N_PAD, D), jnp.float32),
        mesh=mesh,
        scratch_types=[
            pltpu.VMEM((B_CH,), jnp.int32),
            pltpu.VMEM((B_CH,), jnp.int32),
            pltpu.VMEM((B_CH,), jnp.int32),
            pltpu.VMEM((B_CH,), jnp.int32),
            pltpu.VMEM((B_CH,), jnp.int32),
            pltpu.VMEM((B_CH,), jnp.int32),
            pltpu.VMEM((B_CH, D), jnp.float32),
            pltpu.VMEM_SHARED((N_PAD, D), jnp.float32),
            pltpu.SemaphoreType.DMA,
            pltpu.SemaphoreType.DMA,
            pltpu.SemaphoreType.DMA,
        ],
    )
    return k(xw_flat, src, et, dst, zeros)


def _comb_body(p_ref, feat_ref, lw_ref, b_ref, out_ref):
    out_ref[...] = (p_ref[0] + p_ref[1] + b_ref[...]
                    + jnp.dot(feat_ref[...], lw_ref[...],
                              preferred_element_type=jnp.float32))


def _combine(part, feat, loop_weight, bias2d):
    bn = 2000
    nb = N_NODES // bn
    return pl.pallas_call(
        _comb_body,
        grid=(nb,),
        in_specs=[
            pl.BlockSpec((NUM_CORES, bn, D), lambda i: (0, i, 0)),
            pl.BlockSpec((bn, D), lambda i: (i, 0)),
            pl.BlockSpec((D, D), lambda i: (0, 0)),
            pl.BlockSpec((1, D), lambda i: (0, 0)),
        ],
        out_specs=pl.BlockSpec((bn, D), lambda i: (i, 0)),
        out_shape=jax.ShapeDtypeStruct((N_NODES, D), jnp.float32),
    )(part, feat, loop_weight, bias2d)


def kernel(feat, edge_index, etypes, weight, loop_weight, bias):
    npad = E_PAD - N_EDGES
    src = jnp.concatenate(
        [edge_index[0].astype(jnp.int32), jnp.zeros((npad,), jnp.int32)])
    dst = jnp.concatenate(
        [edge_index[1].astype(jnp.int32),
         jnp.full((npad,), N_PAD - 1, jnp.int32)])
    et = jnp.concatenate(
        [etypes.astype(jnp.int32), jnp.zeros((npad,), jnp.int32)])
    zeros = jnp.zeros((N_PAD, D), jnp.float32)

    xw = _make_xw(feat, weight)
    xw_flat = xw.reshape(NUM_RELS * N_NODES, D)
    part = _run_sc(xw_flat, src, et, dst, zeros)
    bias2d = bias.reshape(1, D)
    return _combine(part, feat, loop_weight, bias2d)


# speedup vs baseline: 1.7055x; 1.7055x over previous
import functools

import jax
import jax.numpy as jnp
from jax import lax
from jax.experimental import pallas as pl
from jax.experimental.pallas import tpu as pltpu
from jax.experimental.pallas import tpu_sc as plsc

N_NODES = 10000
N_EDGES = 320000
D = 128
NUM_RELS = 16

NUM_CORES = 2
NUM_SUBCORES = 16
NW = NUM_CORES * NUM_SUBCORES
B_CH = 128
NUM_CHUNKS = N_EDGES // B_CH
CHUNKS_PER_W = -(-NUM_CHUNKS // NW)
N_PAD = 10240
ROWS_PER_SUB = N_PAD // NUM_SUBCORES


def _xw_body(feat_ref, w_ref, xw_ref):
    xw_ref[0] = jnp.dot(feat_ref[...], w_ref[0],
                        preferred_element_type=jnp.float32)


def _make_xw(feat, weight):
    bn = 2000
    nb = N_NODES // bn
    return pl.pallas_call(
        _xw_body,
        grid=(nb, NUM_RELS),
        in_specs=[
            pl.BlockSpec((bn, D), lambda i, r: (i, 0)),
            pl.BlockSpec((1, D, D), lambda i, r: (r, 0, 0)),
        ],
        out_specs=pl.BlockSpec((1, bn, D), lambda i, r: (r, i, 0)),
        out_shape=jax.ShapeDtypeStruct((NUM_RELS, N_NODES, D), jnp.float32),
    )(feat, weight)


def _sc_body(xw_hbm, src_hbm, et_hbm, dst_hbm, zeros_hbm, part_hbm,
             src_v, et_v, dst_v, idx_v, rows_v, acc_shared, sem):
    core = lax.axis_index("c")
    sub = lax.axis_index("s")
    wid = core * NUM_SUBCORES + sub

    r0 = sub * ROWS_PER_SUB
    pltpu.sync_copy(zeros_hbm.at[pl.ds(r0, ROWS_PER_SUB)],
                    acc_shared.at[pl.ds(r0, ROWS_PER_SUB)])
    plsc.subcore_barrier()

    @pl.loop(0, CHUNKS_PER_W)
    def _(j):
        c = j * NW + wid

        @pl.when(c < NUM_CHUNKS)
        def _():
            base = pl.multiple_of(c * B_CH, B_CH)
            pltpu.sync_copy(src_hbm.at[pl.ds(base, B_CH)], src_v)
            pltpu.sync_copy(et_hbm.at[pl.ds(base, B_CH)], et_v)
            pltpu.sync_copy(dst_hbm.at[pl.ds(base, B_CH)], dst_v)
            for k in range(B_CH // 16):
                sl = pl.ds(k * 16, 16)
                idx_v[sl] = et_v[sl] * N_NODES + src_v[sl]
            pltpu.async_copy(xw_hbm.at[idx_v], rows_v, sem).wait()
            pltpu.sync_copy(rows_v, acc_shared.at[dst_v], add=True)

    plsc.subcore_barrier()
    pltpu.sync_copy(acc_shared.at[pl.ds(r0, ROWS_PER_SUB)],
                    part_hbm.at[core, pl.ds(r0, ROWS_PER_SUB)])


def _run_sc(xw_flat, src, et, dst, zeros):
    mesh = plsc.VectorSubcoreMesh(core_axis_name="c", subcore_axis_name="s")
    k = pl.kernel(
        _sc_body,
        out_type=jax.ShapeDtypeStruct((NUM_CORES, N_PAD, D), jnp.float32),
        mesh=mesh,
        scratch_types=[
            pltpu.VMEM((B_CH,), jnp.int32),
            pltpu.VMEM((B_CH,), jnp.int32),
            pltpu.VMEM((B_CH,), jnp.int32),
            pltpu.VMEM((B_CH,), jnp.int32),
            pltpu.VMEM((B_CH, D), jnp.float32),
            pltpu.VMEM_SHARED((N_PAD, D), jnp.float32),
            pltpu.SemaphoreType.DMA,
        ],
    )
    return k(xw_flat, src, et, dst, zeros)


def _comb_body(p_ref, feat_ref, lw_ref, b_ref, out_ref):
    out_ref[...] = (p_ref[0] + p_ref[1] + b_ref[...]
                    + jnp.dot(feat_ref[...], lw_ref[...],
                              preferred_element_type=jnp.float32))


def _combine(part, feat, loop_weight, bias2d):
    bn = 2000
    nb = N_NODES // bn
    return pl.pallas_call(
        _comb_body,
        grid=(nb,),
        in_specs=[
            pl.BlockSpec((NUM_CORES, bn, D), lambda i: (0, i, 0)),
            pl.BlockSpec((bn, D), lambda i: (i, 0)),
            pl.BlockSpec((D, D), lambda i: (0, 0)),
            pl.BlockSpec((1, D), lambda i: (0, 0)),
        ],
        out_specs=pl.BlockSpec((bn, D), lambda i: (i, 0)),
        out_shape=jax.ShapeDtypeStruct((N_NODES, D), jnp.float32),
    )(part, feat, loop_weight, bias2d)


def kernel(feat, edge_index, etypes, weight, loop_weight, bias):
    src = edge_index[0].astype(jnp.int32)
    dst = edge_index[1].astype(jnp.int32)
    et = etypes.astype(jnp.int32)
    zeros = jnp.zeros((N_PAD, D), jnp.float32)

    xw = _make_xw(feat, weight)
    xw_flat = xw.reshape(NUM_RELS * N_NODES, D)
    part = _run_sc(xw_flat, src, et, dst, zeros)
    bias2d = bias.reshape(1, D)
    return _combine(part, feat, loop_weight, bias2d)


# speedup vs baseline: 2.2811x; 1.3375x over previous
import functools

import jax
import jax.numpy as jnp
from jax import lax
from jax.experimental import pallas as pl
from jax.experimental.pallas import tpu as pltpu
from jax.experimental.pallas import tpu_sc as plsc

N_NODES = 10000
N_EDGES = 320000
D = 128
NUM_RELS = 16

NUM_CORES = 2
NUM_SUBCORES = 16
NW = NUM_CORES * NUM_SUBCORES
B_CH = 128
NUM_CHUNKS = N_EDGES // B_CH
CHUNKS_PER_W = -(-NUM_CHUNKS // NW)
N_PAD = 10240
ROWS_PER_SUB = N_PAD // NUM_SUBCORES


def _xw_body(feat_ref, w_ref, xw_ref):
    xw_ref[0] = jnp.dot(feat_ref[...], w_ref[0],
                        preferred_element_type=jnp.float32)


def _make_xw(feat, weight):
    bn = 2000
    nb = N_NODES // bn
    return pl.pallas_call(
        _xw_body,
        grid=(nb, NUM_RELS),
        in_specs=[
            pl.BlockSpec((bn, D), lambda i, r: (i, 0)),
            pl.BlockSpec((1, D, D), lambda i, r: (r, 0, 0)),
        ],
        out_specs=pl.BlockSpec((1, bn, D), lambda i, r: (r, i, 0)),
        out_shape=jax.ShapeDtypeStruct((NUM_RELS, N_NODES, D), jnp.float32),
    )(feat, weight)


def _sc_body(xw_hbm, src_hbm, et_hbm, dst_hbm, zeros_hbm, part_hbm,
             et_a, et_b, dst_a, dst_b, idx_a, idx_b, rows_a, rows_b,
             acc_shared, sem_a, sem_b):
    core = lax.axis_index("c")
    sub = lax.axis_index("s")
    wid = core * NUM_SUBCORES + sub

    r0 = sub * ROWS_PER_SUB
    pltpu.sync_copy(zeros_hbm.at[pl.ds(r0, ROWS_PER_SUB)],
                    acc_shared.at[pl.ds(r0, ROWS_PER_SUB)])
    plsc.subcore_barrier()

    bufs = ((et_a, dst_a, idx_a, rows_a, sem_a),
            (et_b, dst_b, idx_b, rows_b, sem_b))

    def stage(j, b):
        c = j * NW + wid

        @pl.when(c < NUM_CHUNKS)
        def _():
            et, dst, idx, rows, sem = bufs[b]
            base = pl.multiple_of(c * B_CH, B_CH)
            pltpu.sync_copy(src_hbm.at[pl.ds(base, B_CH)], idx)
            pltpu.sync_copy(et_hbm.at[pl.ds(base, B_CH)], et)
            pltpu.sync_copy(dst_hbm.at[pl.ds(base, B_CH)], dst)
            for k in range(B_CH // 16):
                sl = pl.ds(k * 16, 16)
                idx[sl] = et[sl] * N_NODES + idx[sl]
            pltpu.async_copy(xw_hbm.at[idx], rows, sem)

    def drain_scat(j, b):
        c = j * NW + wid

        @pl.when(c < NUM_CHUNKS)
        def _():
            et, dst, idx, rows, sem = bufs[b]
            pltpu.make_async_copy(xw_hbm.at[pl.ds(0, B_CH)], rows, sem).wait()
            pltpu.sync_copy(rows, acc_shared.at[dst], add=True)

    assert CHUNKS_PER_W % 2 == 1
    stage(0, 0)

    @pl.loop(0, CHUNKS_PER_W - 1, step=2)
    def _(j):
        stage(j + 1, 1)
        drain_scat(j, 0)
        stage(j + 2, 0)
        drain_scat(j + 1, 1)

    drain_scat(CHUNKS_PER_W - 1, 0)

    plsc.subcore_barrier()
    pltpu.sync_copy(acc_shared.at[pl.ds(r0, ROWS_PER_SUB)],
                    part_hbm.at[core, pl.ds(r0, ROWS_PER_SUB)])


def _run_sc(xw_flat, src, et, dst, zeros):
    mesh = plsc.VectorSubcoreMesh(core_axis_name="c", subcore_axis_name="s")
    k = pl.kernel(
        _sc_body,
        out_type=jax.ShapeDtypeStruct((NUM_CORES, N_PAD, D), jnp.float32),
        mesh=mesh,
        scratch_types=[
            pltpu.VMEM((B_CH,), jnp.int32),
            pltpu.VMEM((B_CH,), jnp.int32),
            pltpu.VMEM((B_CH,), jnp.int32),
            pltpu.VMEM((B_CH,), jnp.int32),
            pltpu.VMEM((B_CH,), jnp.int32),
            pltpu.VMEM((B_CH,), jnp.int32),
            pltpu.VMEM((B_CH, D), jnp.float32),
            pltpu.VMEM((B_CH, D), jnp.float32),
            pltpu.VMEM_SHARED((N_PAD, D), jnp.float32),
            pltpu.SemaphoreType.DMA,
            pltpu.SemaphoreType.DMA,
        ],
    )
    return k(xw_flat, src, et, dst, zeros)


def _comb_body(p_ref, feat_ref, lw_ref, b_ref, out_ref):
    out_ref[...] = (p_ref[0] + p_ref[1] + b_ref[...]
                    + jnp.dot(feat_ref[...], lw_ref[...],
                              preferred_element_type=jnp.float32))


def _combine(part, feat, loop_weight, bias2d):
    bn = 2000
    nb = N_NODES // bn
    return pl.pallas_call(
        _comb_body,
        grid=(nb,),
        in_specs=[
            pl.BlockSpec((NUM_CORES, bn, D), lambda i: (0, i, 0)),
            pl.BlockSpec((bn, D), lambda i: (i, 0)),
            pl.BlockSpec((D, D), lambda i: (0, 0)),
            pl.BlockSpec((1, D), lambda i: (0, 0)),
        ],
        out_specs=pl.BlockSpec((bn, D), lambda i: (i, 0)),
        out_shape=jax.ShapeDtypeStruct((N_NODES, D), jnp.float32),
    )(part, feat, loop_weight, bias2d)


def kernel(feat, edge_index, etypes, weight, loop_weight, bias):
    src = edge_index[0].astype(jnp.int32)
    dst = edge_index[1].astype(jnp.int32)
    et = etypes.astype(jnp.int32)
    zeros = jnp.zeros((N_PAD, D), jnp.float32)

    xw = _make_xw(feat, weight)
    xw_flat = xw.reshape(NUM_RELS * N_NODES, D)
    part = _run_sc(xw_flat, src, et, dst, zeros)
    bias2d = bias.reshape(1, D)
    return _combine(part, feat, loop_weight, bias2d)


# speedup vs baseline: 2.6612x; 1.1666x over previous
import functools

import jax
import jax.numpy as jnp
from jax import lax
from jax.experimental import pallas as pl
from jax.experimental.pallas import tpu as pltpu
from jax.experimental.pallas import tpu_sc as plsc

N_NODES = 10000
N_EDGES = 320000
D = 128
NUM_RELS = 16

NUM_CORES = 2
NUM_SUBCORES = 16
NW = NUM_CORES * NUM_SUBCORES
B_CH = 128
NUM_CHUNKS = N_EDGES // B_CH
CHUNKS_PER_W = -(-NUM_CHUNKS // NW)
N_PAD = 10240
ROWS_PER_SUB = N_PAD // NUM_SUBCORES


def _xw_body(feat_ref, w_ref, xw_ref):
    xw_ref[0] = jnp.dot(feat_ref[...], w_ref[0],
                        preferred_element_type=jnp.float32)


def _make_xw(feat, weight):
    bn = 2000
    nb = N_NODES // bn
    return pl.pallas_call(
        _xw_body,
        grid=(nb, NUM_RELS),
        in_specs=[
            pl.BlockSpec((bn, D), lambda i, r: (i, 0)),
            pl.BlockSpec((1, D, D), lambda i, r: (r, 0, 0)),
        ],
        out_specs=pl.BlockSpec((1, bn, D), lambda i, r: (r, i, 0)),
        out_shape=jax.ShapeDtypeStruct((NUM_RELS, N_NODES, D), jnp.float32),
    )(feat, weight)


def _sc_body(xw_hbm, src_hbm, et_hbm, dst_hbm, zeros_hbm, part_hbm,
             et_a, et_b, dst_a, dst_b, idx_a, idx_b, rows_a, rows_b,
             acc_shared, sem_a, sem_b, sem_sa, sem_sb):
    core = lax.axis_index("c")
    sub = lax.axis_index("s")
    wid = core * NUM_SUBCORES + sub

    r0 = sub * ROWS_PER_SUB
    pltpu.sync_copy(zeros_hbm.at[pl.ds(r0, ROWS_PER_SUB)],
                    acc_shared.at[pl.ds(r0, ROWS_PER_SUB)])
    plsc.subcore_barrier()

    bufs = ((et_a, dst_a, idx_a, rows_a, sem_a, sem_sa),
            (et_b, dst_b, idx_b, rows_b, sem_b, sem_sb))

    def wait_scat(b):
        et, dst, idx, rows, sem, sem_s = bufs[b]
        pltpu.make_async_copy(rows, acc_shared.at[pl.ds(0, B_CH)],
                              sem_s).wait()

    def stage(j, b, scat_pending=True):
        c = j * NW + wid

        @pl.when(c < NUM_CHUNKS)
        def _():
            et, dst, idx, rows, sem, sem_s = bufs[b]
            base = pl.multiple_of(c * B_CH, B_CH)
            pltpu.sync_copy(src_hbm.at[pl.ds(base, B_CH)], idx)
            pltpu.sync_copy(et_hbm.at[pl.ds(base, B_CH)], et)
            pltpu.sync_copy(dst_hbm.at[pl.ds(base, B_CH)], dst)
            for k in range(B_CH // 16):
                sl = pl.ds(k * 16, 16)
                idx[sl] = et[sl] * N_NODES + idx[sl]
            if scat_pending:
                wait_scat(b)
            pltpu.async_copy(xw_hbm.at[idx], rows, sem)

    def drain_scat(j, b):
        c = j * NW + wid

        @pl.when(c < NUM_CHUNKS)
        def _():
            et, dst, idx, rows, sem, sem_s = bufs[b]
            pltpu.make_async_copy(xw_hbm.at[pl.ds(0, B_CH)], rows, sem).wait()
            pltpu.async_copy(rows, acc_shared.at[dst], sem_s, add=True)

    assert CHUNKS_PER_W % 2 == 1
    stage(0, 0, scat_pending=False)

    stage(1, 1, scat_pending=False)
    drain_scat(0, 0)
    stage(2, 0)
    drain_scat(1, 1)

    @pl.loop(2, CHUNKS_PER_W - 1, step=2)
    def _(j):
        stage(j + 1, 1)
        drain_scat(j, 0)
        stage(j + 2, 0)
        drain_scat(j + 1, 1)

    drain_scat(CHUNKS_PER_W - 1, 0)
    wait_scat(0)
    wait_scat(1)

    plsc.subcore_barrier()
    pltpu.sync_copy(acc_shared.at[pl.ds(r0, ROWS_PER_SUB)],
                    part_hbm.at[core, pl.ds(r0, ROWS_PER_SUB)])


def _run_sc(xw_flat, src, et, dst, zeros):
    mesh = plsc.VectorSubcoreMesh(core_axis_name="c", subcore_axis_name="s")
    k = pl.kernel(
        _sc_body,
        out_type=jax.ShapeDtypeStruct((NUM_CORES, N_PAD, D), jnp.float32),
        mesh=mesh,
        scratch_types=[
            pltpu.VMEM((B_CH,), jnp.int32),
            pltpu.VMEM((B_CH,), jnp.int32),
            pltpu.VMEM((B_CH,), jnp.int32),
            pltpu.VMEM((B_CH,), jnp.int32),
            pltpu.VMEM((B_CH,), jnp.int32),
            pltpu.VMEM((B_CH,), jnp.int32),
            pltpu.VMEM((B_CH, D), jnp.float32),
            pltpu.VMEM((B_CH, D), jnp.float32),
            pltpu.VMEM_SHARED((N_PAD, D), jnp.float32),
            pltpu.SemaphoreType.DMA,
            pltpu.SemaphoreType.DMA,
            pltpu.SemaphoreType.DMA,
            pltpu.SemaphoreType.DMA,
        ],
    )
    return k(xw_flat, src, et, dst, zeros)


def _comb_body(p_ref, feat_ref, lw_ref, b_ref, out_ref):
    out_ref[...] = (p_ref[0] + p_ref[1] + b_ref[...]
                    + jnp.dot(feat_ref[...], lw_ref[...],
                              preferred_element_type=jnp.float32))


def _combine(part, feat, loop_weight, bias2d):
    bn = 2000
    nb = N_NODES // bn
    return pl.pallas_call(
        _comb_body,
        grid=(nb,),
        in_specs=[
            pl.BlockSpec((NUM_CORES, bn, D), lambda i: (0, i, 0)),
            pl.BlockSpec((bn, D), lambda i: (i, 0)),
            pl.BlockSpec((D, D), lambda i: (0, 0)),
            pl.BlockSpec((1, D), lambda i: (0, 0)),
        ],
        out_specs=pl.BlockSpec((bn, D), lambda i: (i, 0)),
        out_shape=jax.ShapeDtypeStruct((N_NODES, D), jnp.float32),
    )(part, feat, loop_weight, bias2d)


def kernel(feat, edge_index, etypes, weight, loop_weight, bias):
    src = edge_index[0].astype(jnp.int32)
    dst = edge_index[1].astype(jnp.int32)
    et = etypes.astype(jnp.int32)
    zeros = jnp.zeros((N_PAD, D), jnp.float32)

    xw = _make_xw(feat, weight)
    xw_flat = xw.reshape(NUM_RELS * N_NODES, D)
    part = _run_sc(xw_flat, src, et, dst, zeros)
    bias2d = bias.reshape(1, D)
    return _combine(part, feat, loop_weight, bias2d)
